# baseline (speedup 1.0000x reference)
import numpy as np

import jax
import jax.numpy as jnp
from jax.experimental import pallas as pl
from jax.experimental.pallas import tpu as pltpu
from jax.sharding import Mesh, PartitionSpec as P

try:
    from jax import shard_map as _shard_map
except ImportError:
    from jax.experimental.shard_map import shard_map as _shard_map

_NCH = 4


def _bcast_copy(w, B, NCH):
    rows, E = w.shape
    CH = rows // NCH

    def body(w_hbm, o_hbm, buf, in_sem, out_sem):
        def in_cp(j):
            return pltpu.make_async_copy(
                w_hbm.at[pl.ds(j * CH, CH), :],
                buf.at[pl.ds(j * CH, CH), :],
                in_sem.at[j],
            )

        def out_cp(j, b):
            return pltpu.make_async_copy(
                buf.at[pl.ds(j * CH, CH), :],
                o_hbm.at[b, pl.ds(j * CH, CH), :],
                out_sem.at[j, b],
            )

        for j in range(NCH):
            in_cp(j).start()
        for j in range(NCH):
            in_cp(j).wait()
            for b in range(B):
                out_cp(j, b).start()
        for j in range(NCH):
            for b in range(B):
                out_cp(j, b).wait()

    return pl.pallas_call(
        body,
        in_specs=[pl.BlockSpec(memory_space=pl.ANY)],
        out_specs=pl.BlockSpec(memory_space=pl.ANY),
        out_shape=jax.ShapeDtypeStruct((B, rows, E), w.dtype),
        scratch_shapes=[
            pltpu.VMEM((rows, E), w.dtype),
            pltpu.SemaphoreType.DMA((NCH,)),
            pltpu.SemaphoreType.DMA((NCH, B)),
        ],
    )(w)


def kernel(inputs, embed_weight):
    B, S = inputs.shape
    devs = jax.devices()
    n_dev = 2 if len(devs) >= 2 else 1
    if n_dev == 2 and S % (2 * _NCH) == 0:
        mesh = Mesh(np.array(devs[:2]), ("x",))
        f = _shard_map(
            lambda w_loc: _bcast_copy(w_loc, B, _NCH),
            mesh=mesh,
            in_specs=P("x", None),
            out_specs=P(None, "x", None),
            check_vma=False,
        )
        return f(embed_weight)
    return _bcast_copy(embed_weight, B, _NCH)


# speedup vs baseline: 8.6439x; 8.6439x over previous
import jax
import jax.numpy as jnp
from jax.experimental import pallas as pl
from jax.experimental.pallas import tpu as pltpu

_NCH = 4


def kernel(inputs, embed_weight):
    B, S = inputs.shape
    E = embed_weight.shape[1]
    NCH = _NCH
    CH = S // NCH

    def body(w_hbm, o_hbm, buf, in_sem, out_sem):
        def in_cp(j):
            return pltpu.make_async_copy(
                w_hbm.at[pl.ds(j * CH, CH), :],
                buf.at[pl.ds(j * CH, CH), :],
                in_sem.at[j],
            )

        def out_cp(j, b):
            return pltpu.make_async_copy(
                buf.at[pl.ds(j * CH, CH), :],
                o_hbm.at[b, pl.ds(j * CH, CH), :],
                out_sem.at[j, b],
            )

        for j in range(NCH):
            in_cp(j).start()
        for j in range(NCH):
            in_cp(j).wait()
            for b in range(B):
                out_cp(j, b).start()
        for j in range(NCH):
            for b in range(B):
                out_cp(j, b).wait()

    out = pl.pallas_call(
        body,
        in_specs=[pl.BlockSpec(memory_space=pl.ANY)],
        out_specs=pl.BlockSpec(memory_space=pl.ANY),
        out_shape=jax.ShapeDtypeStruct((B, S, E), embed_weight.dtype),
        scratch_shapes=[
            pltpu.VMEM((S, E), embed_weight.dtype),
            pltpu.SemaphoreType.DMA((NCH,)),
            pltpu.SemaphoreType.DMA((NCH, B)),
        ],
    )(embed_weight)
    return out
